# baseline (speedup 1.0000x reference)
import jax
import jax.numpy as jnp
from jax.experimental import pallas as pl
from jax.experimental.pallas import tpu as pltpu

HIDDEN = 1024
NQ = 16
NKV = 4
HD = 64
S = 1024
GROUP = NQ // NKV
BQ = 128
NB = S // BQ
MS = GROUP * BQ

_DN = (((1,), (1,)), ((), ()))


def _rope_full(x, cos_t, sin_t):
    r_minus = jnp.roll(x, -HD // 2, axis=1)
    r_plus = jnp.roll(x, HD // 2, axis=1)
    lane = jax.lax.broadcasted_iota(jnp.int32, x.shape, 1)
    rot = jnp.where(lane % HD < HD // 2, -r_minus, r_plus)
    return x * cos_t + rot * sin_t


def _group_kernel(h_ref, wq_ref, wk_ref, wv_ref, cos_ref, sin_ref, o_ref,
                  q_scr, k_scr, va_scr):
    scaling = HD ** (-0.5)
    g = pl.program_id(0)

    @pl.when(g == 0)
    def _project_all():
        h = h_ref[...]
        cos = cos_ref[...]
        sin = sin_ref[...]
        q_full = jax.lax.dot_general(h, wq_ref[...], _DN,
                                     preferred_element_type=jnp.float32)
        k_full = jax.lax.dot_general(h, wk_ref[...], _DN,
                                     preferred_element_type=jnp.float32)
        v_full = jax.lax.dot_general(h, wv_ref[...], _DN,
                                     preferred_element_type=jnp.float32)
        q_full = _rope_full(q_full, jnp.tile(cos, (1, NQ)),
                            jnp.tile(sin, (1, NQ))) * scaling
        k_full = _rope_full(k_full, jnp.tile(cos, (1, NKV)),
                            jnp.tile(sin, (1, NKV)))
        ones = jnp.ones((S, HD), dtype=jnp.float32)
        for gg in range(NKV):
            k_scr[gg] = k_full[:, gg * HD:(gg + 1) * HD]
            va_scr[gg] = jnp.concatenate(
                [v_full[:, gg * HD:(gg + 1) * HD], ones], axis=1)
            for i in range(NB):
                stack = jnp.concatenate(
                    [q_full[i * BQ:(i + 1) * BQ,
                            (gg * GROUP + hh) * HD:(gg * GROUP + hh + 1) * HD]
                     for hh in range(GROUP)], axis=0)
                q_scr[gg, i] = stack

    k = k_scr[g]
    v_aug = va_scr[g]

    rows = jax.lax.broadcasted_iota(jnp.int32, (MS, BQ), 0) % BQ
    cols = jax.lax.broadcasted_iota(jnp.int32, (MS, BQ), 1)
    bias = jnp.where(rows >= cols, 0.0, -1e30).astype(jnp.float32)

    for i in range(NB):
        lo = i * BQ
        qi = q_scr[g, i]
        s_d = jax.lax.dot_general(qi, k[lo:lo + BQ], _DN,
                                  preferred_element_type=jnp.float32)
        s_d = s_d + bias
        if i == 0:
            m = jnp.max(s_d, axis=1, keepdims=True)
            o_aug = jnp.dot(jnp.exp(s_d - m), v_aug[lo:lo + BQ],
                            preferred_element_type=jnp.float32)
        else:
            s_p = jax.lax.dot_general(qi, k[:lo], _DN,
                                      preferred_element_type=jnp.float32)
            m = jnp.maximum(jnp.max(s_p, axis=1, keepdims=True),
                            jnp.max(s_d, axis=1, keepdims=True))
            o_aug = (jnp.dot(jnp.exp(s_p - m), v_aug[:lo],
                             preferred_element_type=jnp.float32)
                     + jnp.dot(jnp.exp(s_d - m), v_aug[lo:lo + BQ],
                               preferred_element_type=jnp.float32))
        o = o_aug[:, :HD] / o_aug[:, HD:]
        for hh in range(GROUP):
            o_ref[hh, lo:lo + BQ, :] = o[hh * BQ:(hh + 1) * BQ]


def kernel(hidden_states, cos, sin, Wq, Wk, Wv):
    h2d = hidden_states[0]
    cos2d = cos[0]
    sin2d = sin[0]

    out = pl.pallas_call(
        _group_kernel,
        grid=(NKV,),
        in_specs=[
            pl.BlockSpec((S, HIDDEN), lambda g: (0, 0)),
            pl.BlockSpec((NQ * HD, HIDDEN), lambda g: (0, 0)),
            pl.BlockSpec((NKV * HD, HIDDEN), lambda g: (0, 0)),
            pl.BlockSpec((NKV * HD, HIDDEN), lambda g: (0, 0)),
            pl.BlockSpec((S, HD), lambda g: (0, 0)),
            pl.BlockSpec((S, HD), lambda g: (0, 0)),
        ],
        out_specs=pl.BlockSpec((GROUP, S, HD), lambda g: (g, 0, 0)),
        out_shape=jax.ShapeDtypeStruct((NQ, S, HD), jnp.float32),
        scratch_shapes=[
            pltpu.VMEM((NKV, NB, MS, HD), jnp.float32),
            pltpu.VMEM((NKV, S, HD), jnp.float32),
            pltpu.VMEM((NKV, S, 2 * HD), jnp.float32),
        ],
    )(h2d, Wq, Wk, Wv, cos2d, sin2d)
    return out


# speedup vs baseline: 1.0085x; 1.0085x over previous
import jax
import jax.numpy as jnp
from jax.experimental import pallas as pl
from jax.experimental.pallas import tpu as pltpu

HIDDEN = 1024
NQ = 16
NKV = 4
HD = 64
S = 1024
GROUP = NQ // NKV
BQ = 256
NB = S // BQ
MS = GROUP * BQ

_DN = (((1,), (1,)), ((), ()))


def _rope_full(x, cos_t, sin_t):
    r_minus = jnp.roll(x, -HD // 2, axis=1)
    r_plus = jnp.roll(x, HD // 2, axis=1)
    lane = jax.lax.broadcasted_iota(jnp.int32, x.shape, 1)
    rot = jnp.where(lane % HD < HD // 2, -r_minus, r_plus)
    return x * cos_t + rot * sin_t


def _group_kernel(h_ref, wq_ref, wk_ref, wv_ref, cos_ref, sin_ref, o_ref,
                  q_scr, k_scr, va_scr):
    scaling = HD ** (-0.5)
    g = pl.program_id(0)

    @pl.when(g == 0)
    def _project_all():
        h = h_ref[...]
        cos = cos_ref[...]
        sin = sin_ref[...]
        q_full = jax.lax.dot_general(h, wq_ref[...], _DN,
                                     preferred_element_type=jnp.float32)
        k_full = jax.lax.dot_general(h, wk_ref[...], _DN,
                                     preferred_element_type=jnp.float32)
        v_full = jax.lax.dot_general(h, wv_ref[...], _DN,
                                     preferred_element_type=jnp.float32)
        q_full = _rope_full(q_full, jnp.tile(cos, (1, NQ)),
                            jnp.tile(sin, (1, NQ))) * scaling
        k_full = _rope_full(k_full, jnp.tile(cos, (1, NKV)),
                            jnp.tile(sin, (1, NKV)))
        ones = jnp.ones((S, HD), dtype=jnp.float32)
        for gg in range(NKV):
            k_scr[gg] = k_full[:, gg * HD:(gg + 1) * HD]
            va_scr[gg] = jnp.concatenate(
                [v_full[:, gg * HD:(gg + 1) * HD], ones], axis=1)
            for i in range(NB):
                stack = jnp.concatenate(
                    [q_full[i * BQ:(i + 1) * BQ,
                            (gg * GROUP + hh) * HD:(gg * GROUP + hh + 1) * HD]
                     for hh in range(GROUP)], axis=0)
                q_scr[gg, i] = stack

    k = k_scr[g]
    v_aug = va_scr[g]

    rows = jax.lax.broadcasted_iota(jnp.int32, (MS, BQ), 0) % BQ
    cols = jax.lax.broadcasted_iota(jnp.int32, (MS, BQ), 1)
    bias = jnp.where(rows >= cols, 0.0, -1e30).astype(jnp.float32)

    for i in range(NB):
        lo = i * BQ
        qi = q_scr[g, i]
        s_d = jax.lax.dot_general(qi, k[lo:lo + BQ], _DN,
                                  preferred_element_type=jnp.float32)
        s_d = s_d + bias
        if i == 0:
            m = jnp.max(s_d, axis=1, keepdims=True)
            o_aug = jnp.dot(jnp.exp(s_d - m), v_aug[lo:lo + BQ],
                            preferred_element_type=jnp.float32)
        else:
            s_p = jax.lax.dot_general(qi, k[:lo], _DN,
                                      preferred_element_type=jnp.float32)
            m = jnp.maximum(jnp.max(s_p, axis=1, keepdims=True),
                            jnp.max(s_d, axis=1, keepdims=True))
            o_aug = (jnp.dot(jnp.exp(s_p - m), v_aug[:lo],
                             preferred_element_type=jnp.float32)
                     + jnp.dot(jnp.exp(s_d - m), v_aug[lo:lo + BQ],
                               preferred_element_type=jnp.float32))
        o = o_aug[:, :HD] / o_aug[:, HD:]
        for hh in range(GROUP):
            o_ref[hh, lo:lo + BQ, :] = o[hh * BQ:(hh + 1) * BQ]


def kernel(hidden_states, cos, sin, Wq, Wk, Wv):
    h2d = hidden_states[0]
    cos2d = cos[0]
    sin2d = sin[0]

    out = pl.pallas_call(
        _group_kernel,
        grid=(NKV,),
        in_specs=[
            pl.BlockSpec((S, HIDDEN), lambda g: (0, 0)),
            pl.BlockSpec((NQ * HD, HIDDEN), lambda g: (0, 0)),
            pl.BlockSpec((NKV * HD, HIDDEN), lambda g: (0, 0)),
            pl.BlockSpec((NKV * HD, HIDDEN), lambda g: (0, 0)),
            pl.BlockSpec((S, HD), lambda g: (0, 0)),
            pl.BlockSpec((S, HD), lambda g: (0, 0)),
        ],
        out_specs=pl.BlockSpec((GROUP, S, HD), lambda g: (g, 0, 0)),
        out_shape=jax.ShapeDtypeStruct((NQ, S, HD), jnp.float32),
        scratch_shapes=[
            pltpu.VMEM((NKV, NB, MS, HD), jnp.float32),
            pltpu.VMEM((NKV, S, HD), jnp.float32),
            pltpu.VMEM((NKV, S, 2 * HD), jnp.float32),
        ],
    )(h2d, Wq, Wk, Wv, cos2d, sin2d)
    return out
